# baseline (speedup 1.0000x reference)
import functools

import jax
import jax.numpy as jnp
from jax import lax
from jax.experimental import pallas as pl
from jax.experimental.pallas import tpu as pltpu
from jax.experimental.pallas import tpu_sc as plsc

TEMP = 0.05
INV_TEMP = 20.0
BATCH = 1024
NUM_FEATURES = 64
NUM_SAMPLES = 100000
C_TILE = 2048
NUM_TILES = (NUM_SAMPLES + C_TILE - 1) // C_TILE

NEG_BIG = -1e30

_SC_INFO = plsc.get_sparse_core_info()
_NC, _NS = _SC_INFO.num_cores, _SC_INFO.num_subcores
_NW = _NC * _NS
_B_PER_W = BATCH // _NW


def _sc_gather(feat_hbm, tgt_hbm, out_hbm, idx_v, rows_v, sem):
    wid = lax.axis_index("s") * _NC + lax.axis_index("c")
    base = wid * _B_PER_W
    pltpu.sync_copy(tgt_hbm.at[pl.ds(base, _B_PER_W)], idx_v)
    pltpu.async_copy(feat_hbm.at[idx_v], rows_v, sem).wait()
    pltpu.sync_copy(rows_v, out_hbm.at[pl.ds(base, _B_PER_W)])


_sc_gather_call = functools.partial(
    pl.kernel,
    mesh=plsc.VectorSubcoreMesh(core_axis_name="c", subcore_axis_name="s"),
    out_type=jax.ShapeDtypeStruct((BATCH, 2 * NUM_FEATURES), jnp.float32),
    scratch_types=[
        pltpu.VMEM((_B_PER_W,), jnp.int32),
        pltpu.VMEM((_B_PER_W, 2 * NUM_FEATURES), jnp.float32),
        pltpu.SemaphoreType.DMA,
    ],
)(_sc_gather)


def _ce_kernel(inputs_ref, g_ref, tgt_ref, feat_ref, out_ref, loss_ref,
               xn_ref, s_ref):
    i = pl.program_id(0)

    @pl.when(i == 0)
    def _init():
        x = inputs_ref[...]
        norm = jnp.sqrt(jnp.sum(x * x, axis=1, keepdims=True))
        xn_ref[...] = x / jnp.maximum(norm, 1e-12)
        s_ref[...] = jnp.zeros((BATCH, 1), jnp.float32)

    xn = xn_ref[...]
    logits = jax.lax.dot_general(
        xn, feat_ref[...],
        dimension_numbers=(((1,), (1,)), ((), ())),
        preferred_element_type=jnp.float32,
    ) * INV_TEMP
    out_ref[...] = logits

    @pl.when(i < NUM_TILES - 1)
    def _acc():
        s_ref[...] += jnp.sum(jnp.exp(logits - INV_TEMP), axis=1, keepdims=True)

    @pl.when(i == NUM_TILES - 1)
    def _fin():
        cols = (i * C_TILE
                + jax.lax.broadcasted_iota(jnp.int32, (BATCH, C_TILE), 1))
        masked = jnp.where(cols < NUM_SAMPLES, logits, NEG_BIG)
        s = s_ref[...] + jnp.sum(jnp.exp(masked - INV_TEMP), axis=1,
                                 keepdims=True)
        lse = INV_TEMP + jnp.log(s)
        odd = (tgt_ref[...] % 2) == 1
        g = jnp.where(odd, g_ref[:, NUM_FEATURES:], g_ref[:, :NUM_FEATURES])
        picked = jnp.sum(xn * g, axis=1, keepdims=True) * INV_TEMP
        loss = -jnp.mean(picked - lse)
        loss = jnp.where(jnp.isnan(loss), jnp.float32(0.0), loss)
        loss_ref[...] = jnp.reshape(loss, (1, 1))


@jax.jit
def _run(inputs, targets, features):
    tgt = targets.astype(jnp.int32)
    feat2 = features.reshape(NUM_SAMPLES // 2, 2 * NUM_FEATURES)
    gathered = _sc_gather_call(feat2, tgt // 2)
    out, loss = pl.pallas_call(
        _ce_kernel,
        grid=(NUM_TILES,),
        in_specs=[
            pl.BlockSpec((BATCH, NUM_FEATURES), lambda i: (0, 0)),
            pl.BlockSpec((BATCH, 2 * NUM_FEATURES), lambda i: (0, 0)),
            pl.BlockSpec((BATCH, 1), lambda i: (0, 0)),
            pl.BlockSpec((C_TILE, NUM_FEATURES), lambda i: (i, 0)),
        ],
        out_specs=[
            pl.BlockSpec((BATCH, C_TILE), lambda i: (0, i)),
            pl.BlockSpec((1, 1), lambda i: (0, 0)),
        ],
        out_shape=[
            jax.ShapeDtypeStruct((BATCH, NUM_SAMPLES), jnp.float32),
            jax.ShapeDtypeStruct((1, 1), jnp.float32),
        ],
        scratch_shapes=[
            pltpu.VMEM((BATCH, NUM_FEATURES), jnp.float32),
            pltpu.VMEM((BATCH, 1), jnp.float32),
        ],
    )(inputs, gathered, tgt.reshape(BATCH, 1), features)
    return loss[0, 0], out


def kernel(inputs, targets, features):
    loss, out = _run(inputs, targets, features)
    return (loss, out)
